# Initial kernel scaffold; baseline (speedup 1.0000x reference)
import functools

import jax
import jax.numpy as jnp
from jax.experimental import pallas as pl
from jax.experimental.pallas import tpu as pltpu

N = 10000
NPAD = 10240
S = 2048
H = 256
K = 64

_HI_INIT = 0x41000000
_PAD_KEY = 0x7F000000


def _embed_pieces(col, eff):
    om = 1.0 / (10000.0 ** (jnp.arange(eff, dtype=jnp.float32) / eff))
    ph = col * om[None, :]
    return [jnp.sin(ph), jnp.cos(ph)]


def _stage_a_body(idx_ref, xpos8_ref, xt8_ref, keys_ref, thr_ref, y8_ref):
    q = idx_ref.shape[0]
    idx = idx_ref[:]
    cols = jax.lax.broadcasted_iota(jnp.int32, (q, NPAD), 1)
    oh = (cols == idx).astype(jnp.float32)
    y8 = jnp.dot(oh, xpos8_ref[:], preferred_element_type=jnp.float32)
    y8_ref[:] = y8
    xt8 = xt8_ref[:]
    xn2 = jnp.sum(xt8 * xt8, axis=0, keepdims=True)
    yn2 = jnp.sum(y8 * y8, axis=1, keepdims=True)
    g = jnp.dot(y8, xt8, preferred_element_type=jnp.float32)
    d2 = jnp.maximum(yn2 + xn2 - 2.0 * g, 0.0)
    keys = jax.lax.bitcast_convert_type(d2, jnp.int32)
    pad = cols >= N
    keys = jnp.where(pad, _PAD_KEY, keys)
    keys_ref[:] = keys

    def body(_, lohi):
        lo, hi = lohi
        mid = lo + (hi - lo) // 2
        cnt = jnp.sum((keys <= mid).astype(jnp.int32), axis=1, keepdims=True)
        take = cnt >= K
        return jnp.where(take, lo, mid), jnp.where(take, mid, hi)

    lo0 = jnp.full((q, 1), -1, jnp.int32)
    hi0 = jnp.full((q, 1), _HI_INIT, jnp.int32)
    _, hi = jax.lax.fori_loop(0, 31, body, (lo0, hi0))
    thr_ref[:] = hi


def _stage_a(supernode_idx, xpos8, xt8, q_blk, interpret=False):
    grid = S // q_blk
    return pl.pallas_call(
        _stage_a_body,
        grid=(grid,),
        in_specs=[
            pl.BlockSpec((q_blk, 1), lambda i: (i, 0)),
            pl.BlockSpec((NPAD, 8), lambda i: (0, 0)),
            pl.BlockSpec((8, NPAD), lambda i: (0, 0)),
        ],
        out_specs=[
            pl.BlockSpec((q_blk, NPAD), lambda i: (i, 0)),
            pl.BlockSpec((q_blk, 1), lambda i: (i, 0)),
            pl.BlockSpec((q_blk, 8), lambda i: (i, 0)),
        ],
        out_shape=[
            jax.ShapeDtypeStruct((S, NPAD), jnp.int32),
            jax.ShapeDtypeStruct((S, 1), jnp.int32),
            jax.ShapeDtypeStruct((S, 8), jnp.float32),
        ],
        interpret=interpret,
    )(supernode_idx.reshape(S, 1).astype(jnp.int32), xpos8, xt8)


def _stage_c_body(ef_ref, y8_ref, w1_ref, b1_ref, w2_ref, b2_ref, wp_ref,
                  bp_ref, out_ref):
    qk = ef_ref.shape[0]
    q = qk // K
    ef = ef_ref[:]
    mag = jnp.sqrt(jnp.maximum(ef[:, 3:4], 0.0))
    pieces = []
    for c in range(3):
        pieces += _embed_pieces(ef[:, c:c + 1], 32)
    pieces += _embed_pieces(mag, 32)
    relemb = jnp.concatenate(pieces, axis=1)
    h = jax.nn.gelu(
        jnp.dot(relemb, w1_ref[:], preferred_element_type=jnp.float32)
        + b1_ref[:])
    hm = jnp.sum(h.reshape(q, K, H), axis=1) * (1.0 / K)
    agg = jnp.dot(hm, w2_ref[:], preferred_element_type=jnp.float32) + b2_ref[:]
    y = y8_ref[:]
    spieces = []
    for c in range(3):
        spieces += _embed_pieces(y[:, c:c + 1], 42)
    spieces.append(jnp.zeros((q, 4), jnp.float32))
    spe = jnp.concatenate(spieces, axis=1)
    out = (jnp.dot(agg, wp_ref[:H], preferred_element_type=jnp.float32)
           + jnp.dot(spe, wp_ref[H:], preferred_element_type=jnp.float32)
           + bp_ref[:])
    out_ref[:] = out


def _stage_c(ef, y8, w1, b1, w2, b2, wp, bp, q_blk, interpret=False):
    grid = S // q_blk
    return pl.pallas_call(
        _stage_c_body,
        grid=(grid,),
        in_specs=[
            pl.BlockSpec((q_blk * K, 4), lambda i: (i, 0)),
            pl.BlockSpec((q_blk, 8), lambda i: (i, 0)),
            pl.BlockSpec((H, H), lambda i: (0, 0)),
            pl.BlockSpec((1, H), lambda i: (0, 0)),
            pl.BlockSpec((H, H), lambda i: (0, 0)),
            pl.BlockSpec((1, H), lambda i: (0, 0)),
            pl.BlockSpec((2 * H, H), lambda i: (0, 0)),
            pl.BlockSpec((1, H), lambda i: (0, 0)),
        ],
        out_specs=pl.BlockSpec((q_blk, H), lambda i: (i, 0)),
        out_shape=jax.ShapeDtypeStruct((S, H), jnp.float32),
        interpret=interpret,
    )(ef, y8, w1, b1.reshape(1, H), w2, b2.reshape(1, H), wp,
      bp.reshape(1, H))


def _stage_b_jnp(keys, thr, y8, input_pos):
    mask = keys[:, :N] <= thr
    src = jax.vmap(lambda m: jnp.nonzero(m, size=K, fill_value=0)[0])(mask)
    src_pos = input_pos[src.reshape(-1)]
    dst_pos = jnp.repeat(y8[:, :3], K, axis=0)
    dist = dst_pos - src_pos
    d2 = jnp.sum(dist * dist, axis=1, keepdims=True)
    return jnp.concatenate([dist, d2], axis=1)


def kernel(input_pos, supernode_idx, W1, b1, W2, b2, Wp, bp):
    xpos8 = jnp.pad(input_pos, ((0, NPAD - N), (0, 5)), constant_values=0.0)
    xt8 = xpos8.T
    keys, thr, y8 = _stage_a(supernode_idx, xpos8, xt8, q_blk=128)
    ef = _stage_b_jnp(keys, thr, y8, input_pos)
    return _stage_c(ef, y8, W1, b1, W2, b2, Wp, bp, q_blk=64)


# speedup vs baseline: 1.0247x; 1.0247x over previous
import functools

import jax
import jax.numpy as jnp
import numpy as np
from jax.experimental import pallas as pl
from jax.experimental.pallas import tpu as pltpu

N = 10000
NPAD = 10240
S = 2048
H = 256
K = 64

_HI_INIT = 0x41000000
_PAD_KEY = 0x7F000000


def _embed_pieces(col, eff):
    t = jax.lax.broadcasted_iota(jnp.int32, (1, eff), 1).astype(jnp.float32)
    om = jnp.exp(t * (-np.log(10000.0) / eff))
    ph = col * om
    return [jnp.sin(ph), jnp.cos(ph)]


def _stage_a_body(idx_ref, xpos8_ref, xt8_ref, keys_ref, thr_ref, y8_ref):
    q = idx_ref.shape[0]
    idx = idx_ref[:]
    cols = jax.lax.broadcasted_iota(jnp.int32, (q, NPAD), 1)
    oh = (cols == idx).astype(jnp.float32)
    y8 = jnp.dot(oh, xpos8_ref[:], preferred_element_type=jnp.float32)
    y8_ref[:] = y8
    xt8 = xt8_ref[:]
    xn2 = jnp.sum(xt8 * xt8, axis=0, keepdims=True)
    yn2 = jnp.sum(y8 * y8, axis=1, keepdims=True)
    g = jnp.dot(y8, xt8, preferred_element_type=jnp.float32)
    d2 = jnp.maximum(yn2 + xn2 - 2.0 * g, 0.0)
    keys = jax.lax.bitcast_convert_type(d2, jnp.int32)
    pad = cols >= N
    keys = jnp.where(pad, _PAD_KEY, keys)
    keys_ref[:] = keys

    def body(_, lohi):
        lo, hi = lohi
        mid = lo + (hi - lo) // 2
        cnt = jnp.sum((keys <= mid).astype(jnp.int32), axis=1, keepdims=True)
        take = cnt >= K
        return jnp.where(take, lo, mid), jnp.where(take, mid, hi)

    lo0 = jnp.full((q, 1), -1, jnp.int32)
    hi0 = jnp.full((q, 1), _HI_INIT, jnp.int32)
    _, hi = jax.lax.fori_loop(0, 31, body, (lo0, hi0))
    thr_ref[:] = hi


def _stage_a(supernode_idx, xpos8, xt8, q_blk, interpret=False):
    grid = S // q_blk
    return pl.pallas_call(
        _stage_a_body,
        grid=(grid,),
        in_specs=[
            pl.BlockSpec((q_blk, 1), lambda i: (i, 0)),
            pl.BlockSpec((NPAD, 8), lambda i: (0, 0)),
            pl.BlockSpec((8, NPAD), lambda i: (0, 0)),
        ],
        out_specs=[
            pl.BlockSpec((q_blk, NPAD), lambda i: (i, 0)),
            pl.BlockSpec((q_blk, 1), lambda i: (i, 0)),
            pl.BlockSpec((q_blk, 8), lambda i: (i, 0)),
        ],
        out_shape=[
            jax.ShapeDtypeStruct((S, NPAD), jnp.int32),
            jax.ShapeDtypeStruct((S, 1), jnp.int32),
            jax.ShapeDtypeStruct((S, 8), jnp.float32),
        ],
        interpret=interpret,
    )(supernode_idx.reshape(S, 1).astype(jnp.int32), xpos8, xt8)


def _stage_c_body(ef_ref, y8_ref, w1_ref, b1_ref, w2_ref, b2_ref, wp_ref,
                  bp_ref, out_ref):
    qk = ef_ref.shape[0]
    q = qk // K
    ef = ef_ref[:]
    mag = jnp.sqrt(jnp.maximum(ef[:, 3:4], 0.0))
    pieces = []
    for c in range(3):
        pieces += _embed_pieces(ef[:, c:c + 1], 32)
    pieces += _embed_pieces(mag, 32)
    relemb = jnp.concatenate(pieces, axis=1)
    h = jax.nn.gelu(
        jnp.dot(relemb, w1_ref[:], preferred_element_type=jnp.float32)
        + b1_ref[:])
    hm = jnp.sum(h.reshape(q, K, H), axis=1) * (1.0 / K)
    agg = jnp.dot(hm, w2_ref[:], preferred_element_type=jnp.float32) + b2_ref[:]
    y = y8_ref[:]
    spieces = []
    for c in range(3):
        spieces += _embed_pieces(y[:, c:c + 1], 42)
    spieces.append(jnp.zeros((q, 4), jnp.float32))
    spe = jnp.concatenate(spieces, axis=1)
    out = (jnp.dot(agg, wp_ref[:H], preferred_element_type=jnp.float32)
           + jnp.dot(spe, wp_ref[H:], preferred_element_type=jnp.float32)
           + bp_ref[:])
    out_ref[:] = out


def _stage_c(ef, y8, w1, b1, w2, b2, wp, bp, q_blk, interpret=False):
    grid = S // q_blk
    return pl.pallas_call(
        _stage_c_body,
        grid=(grid,),
        in_specs=[
            pl.BlockSpec((q_blk * K, 4), lambda i: (i, 0)),
            pl.BlockSpec((q_blk, 8), lambda i: (i, 0)),
            pl.BlockSpec((H, H), lambda i: (0, 0)),
            pl.BlockSpec((1, H), lambda i: (0, 0)),
            pl.BlockSpec((H, H), lambda i: (0, 0)),
            pl.BlockSpec((1, H), lambda i: (0, 0)),
            pl.BlockSpec((2 * H, H), lambda i: (0, 0)),
            pl.BlockSpec((1, H), lambda i: (0, 0)),
        ],
        out_specs=pl.BlockSpec((q_blk, H), lambda i: (i, 0)),
        out_shape=jax.ShapeDtypeStruct((S, H), jnp.float32),
        interpret=interpret,
    )(ef, y8, w1, b1.reshape(1, H), w2, b2.reshape(1, H), wp,
      bp.reshape(1, H))


def _stage_b_jnp(keys, thr, y8, input_pos):
    mask = keys[:, :N] <= thr
    src = jax.vmap(lambda m: jnp.nonzero(m, size=K, fill_value=0)[0])(mask)
    src_pos = input_pos[src.reshape(-1)]
    dst_pos = jnp.repeat(y8[:, :3], K, axis=0)
    dist = dst_pos - src_pos
    d2 = jnp.sum(dist * dist, axis=1, keepdims=True)
    return jnp.concatenate([dist, d2], axis=1)


def kernel(input_pos, supernode_idx, W1, b1, W2, b2, Wp, bp):
    xpos8 = jnp.pad(input_pos, ((0, NPAD - N), (0, 5)), constant_values=0.0)
    xt8 = xpos8.T
    keys, thr, y8 = _stage_a(supernode_idx, xpos8, xt8, q_blk=128)
    ef = _stage_b_jnp(keys, thr, y8, input_pos)
    return _stage_c(ef, y8, W1, b1, W2, b2, Wp, bp, q_blk=64)
